# baseline (speedup 1.0000x reference)
import functools

import jax
import jax.numpy as jnp
from jax import lax
from jax.experimental import pallas as pl
from jax.experimental.pallas import tpu as pltpu
from jax.experimental.pallas import tpu_sc as plsc

_B = 4
_S = 4096
_D = 1024
_NC = 2
_NS = 16
_NW = _NC * _NS
_PPW = _S // _NW
_C = 8
_NCHUNK = _PPW // _C
_NBUF = 3
_NOUT = -(-_NCHUNK // _NBUF)
_LANES = 16


def _body(tokens_hbm, table_hbm, pos_hbm, out_hbm, idx_v, pos_v, rows_v,
          si0, si1, si2, so0, so1, so2):
    wid = lax.axis_index("s") * _NC + lax.axis_index("c")
    p0 = wid * _PPW
    sem_in = (si0, si1, si2)
    sem_out = (so0, so1, so2)

    pltpu.sync_copy(tokens_hbm.at[pl.ds(wid * _NCHUNK, _NCHUNK)], idx_v)

    def issue_in(ci, k):
        s0 = p0 + ci * _C
        pltpu.async_copy(pos_hbm.at[pl.ds(s0, _C)], pos_v.at[k], sem_in[k])
        pltpu.async_copy(
            table_hbm.at[idx_v.at[ci]],
            rows_v.at[k],
            sem_in[k],
        )

    def wait_in(k):
        pltpu.make_async_copy(pos_hbm.at[pl.ds(0, _C)], pos_v.at[k],
                              sem_in[k]).wait()
        pltpu.make_async_copy(table_hbm.at[pl.ds(0, _B * _C)], rows_v.at[k],
                              sem_in[k]).wait()

    def issue_out(ci, k):
        s0 = p0 + ci * _C
        for b in range(_B):
            pltpu.async_copy(
                rows_v.at[k, pl.ds(b * _C, _C)],
                out_hbm.at[b, pl.ds(s0, _C)],
                sem_out[k],
            )

    def wait_out(k):
        for _ in range(_B):
            pltpu.make_async_copy(pos_hbm.at[pl.ds(0, _C)], pos_v.at[k],
                                  sem_out[k]).wait()

    def add_chunk(k):
        @plsc.parallel_loop(0, _C * (_D // _LANES), unroll=4)
        def _(t):
            c = t >> 6
            off = (t & (_D // _LANES - 1)) * _LANES
            pv = pos_v[k, c, pl.ds(off, _LANES)]
            for b in range(_B):
                r = b * _C + c
                rows_v[k, r, pl.ds(off, _LANES)] = (
                    rows_v[k, r, pl.ds(off, _LANES)] + pv
                )

    for k in range(_NBUF - 1):
        issue_in(k, k)

    def ring(i, carry):
        for k in range(_NBUF):
            ci = i * _NBUF + k
            kprev = (k + _NBUF - 1) % _NBUF
            if k == 0:
                @pl.when(i < _NOUT - 1)
                def _():
                    @pl.when(i > 0)
                    def _():
                        wait_out(kprev)
                    issue_in(ci + _NBUF - 1, kprev)
            else:
                @pl.when(ci + _NBUF - 1 < _NCHUNK)
                def _():
                    wait_out(kprev)
                    issue_in(ci + _NBUF - 1, kprev)

            @pl.when(ci < _NCHUNK)
            def _():
                wait_in(k)
                add_chunk(k)
                issue_out(ci, k)
        return carry

    lax.fori_loop(0, _NOUT, ring, 0)
    for k in range(_NBUF):
        wait_out(k)


@jax.jit
def _emb(tokens, table, pos2d):
    mesh = plsc.VectorSubcoreMesh(core_axis_name="c", subcore_axis_name="s")
    return pl.kernel(
        _body,
        out_type=jax.ShapeDtypeStruct((_B, _S, _D), jnp.float32),
        mesh=mesh,
        compiler_params=pltpu.CompilerParams(use_tc_tiling_on_sc=False),
        scratch_types=[
            pltpu.VMEM((_NCHUNK, _B * _C), jnp.int32),
            pltpu.VMEM((_NBUF, _C, _D), jnp.float32),
            pltpu.VMEM((_NBUF, _B * _C, _D), jnp.float32),
        ] + [pltpu.SemaphoreType.DMA] * (2 * _NBUF),
    )(tokens, table, pos2d)


def kernel(tokens, table, pos):
    tk = tokens.astype(jnp.int32).reshape(_B, _S // _C, _C)
    tk = tk.transpose(1, 0, 2).reshape(_S // _C, _B * _C)
    pos2d = pos.reshape(pos.shape[1], pos.shape[2])[: tokens.shape[1]]
    return _emb(tk, table, pos2d)


# speedup vs baseline: 6.0250x; 6.0250x over previous
import functools

import jax
import jax.numpy as jnp
from jax import lax
from jax.experimental import pallas as pl
from jax.experimental.pallas import tpu as pltpu
from jax.experimental.pallas import tpu_sc as plsc

_B = 4
_S = 4096
_D = 1024
_NC = 2
_NS = 16
_NW = _NC * _NS
_PPW = _S // _NW
_C = 8
_NCHUNK = _PPW // _C
_NBUF = 3
_NOUT = -(-_NCHUNK // _NBUF)
_LANES = 16


def _body(tokens_hbm, table_hbm, pos_hbm, out_hbm, idx_v, pos_v, rows_v,
          si0, si1, si2, so0, so1, so2):
    wid = lax.axis_index("s") * _NC + lax.axis_index("c")
    p0 = wid * _PPW
    sem_in = (si0, si1, si2)
    sem_out = (so0, so1, so2)

    pltpu.sync_copy(tokens_hbm.at[pl.ds(wid * _NCHUNK, _NCHUNK)], idx_v)

    def issue_in(ci, k):
        s0 = p0 + ci * _C
        pltpu.async_copy(pos_hbm.at[pl.ds(s0, _C)], pos_v.at[k], sem_in[k])
        pltpu.async_copy(
            table_hbm.at[idx_v.at[ci]],
            rows_v.at[k],
            sem_in[k],
        )

    def wait_in(k):
        pltpu.make_async_copy(pos_hbm.at[pl.ds(0, _C)], pos_v.at[k],
                              sem_in[k]).wait()
        pltpu.make_async_copy(table_hbm.at[pl.ds(0, _B * _C)], rows_v.at[k],
                              sem_in[k]).wait()

    def issue_out(ci, k):
        s0 = p0 + ci * _C
        for b in range(_B):
            pltpu.async_copy(
                rows_v.at[k, pl.ds(b * _C, _C)],
                out_hbm.at[b, pl.ds(s0, _C)],
                sem_out[k],
            )

    def wait_out(k):
        for _ in range(_B):
            pltpu.make_async_copy(pos_hbm.at[pl.ds(0, _C)], pos_v.at[k],
                                  sem_out[k]).wait()

    def add_chunk(k):
        @plsc.parallel_loop(0, _C * (_D // _LANES), unroll=4)
        def _(t):
            c = t >> 6
            off = (t & (_D // _LANES - 1)) * _LANES
            pv = pos_v[k, c, pl.ds(off, _LANES)]
            for b in range(_B):
                r = b * _C + c
                rows_v[k, r, pl.ds(off, _LANES)] = (
                    rows_v[k, r, pl.ds(off, _LANES)] + pv
                )

    for k in range(_NBUF - 1):
        issue_in(k, k)

    def ring(i, carry):
        for k in range(_NBUF):
            ci = i * _NBUF + k
            kprev = (k + _NBUF - 1) % _NBUF
            if k == 0:
                @pl.when(i < _NOUT - 1)
                def _():
                    @pl.when(i > 0)
                    def _():
                        wait_out(kprev)
                    issue_in(ci + _NBUF - 1, kprev)
            else:
                @pl.when(ci + _NBUF - 1 < _NCHUNK)
                def _():
                    wait_out(kprev)
                    issue_in(ci + _NBUF - 1, kprev)

            @pl.when(ci < _NCHUNK)
            def _():
                wait_in(k)
                add_chunk(k)
                issue_out(ci, k)
        return carry

    lax.fori_loop(0, _NOUT, ring, 0)
    for k in range(_NBUF):
        wait_out(k)


@jax.jit
def _emb(tokens, table, pos2d):
    mesh = plsc.VectorSubcoreMesh(core_axis_name="c", subcore_axis_name="s")
    return pl.kernel(
        _body,
        out_type=jax.ShapeDtypeStruct((_B, _S, _D), jnp.float32),
        mesh=mesh,
        scratch_types=[
            pltpu.VMEM((_NCHUNK, _B * _C), jnp.int32),
            pltpu.VMEM((_NBUF, _C, _D), jnp.float32),
            pltpu.VMEM((_NBUF, _B * _C, _D), jnp.float32),
        ] + [pltpu.SemaphoreType.DMA] * (2 * _NBUF),
    )(tokens, table, pos2d)


def kernel(tokens, table, pos):
    tk = tokens.astype(jnp.int32).reshape(_B, _S // _C, _C)
    tk = tk.transpose(1, 0, 2).reshape(_S // _C, _B * _C)
    pos2d = pos.reshape(pos.shape[1], pos.shape[2])[: tokens.shape[1]]
    return _emb(tk, table, pos2d)


# speedup vs baseline: 6.0282x; 1.0005x over previous
import functools

import jax
import jax.numpy as jnp
from jax import lax
from jax.experimental import pallas as pl
from jax.experimental.pallas import tpu as pltpu
from jax.experimental.pallas import tpu_sc as plsc

_B = 4
_S = 4096
_D = 1024
_NC = 2
_NS = 16
_NW = _NC * _NS
_PPW = _S // _NW
_C = 8
_NCHUNK = _PPW // _C
_NBUF = 3
_NOUT = -(-_NCHUNK // _NBUF)
_LANES = 16


def _body(tokens_hbm, table_hbm, pos_hbm, out_hbm, idx_v, pos_v, rows_v,
          si0, si1, si2, so0, so1, so2):
    wid = lax.axis_index("s") * _NC + lax.axis_index("c")
    p0 = wid * _PPW
    sem_in = (si0, si1, si2)
    sem_out = (so0, so1, so2)

    pltpu.sync_copy(tokens_hbm.at[pl.ds(wid * _NCHUNK, _NCHUNK)], idx_v)

    def issue_in(ci, k):
        s0 = p0 + ci * _C
        pltpu.async_copy(pos_hbm.at[pl.ds(s0, _C)], pos_v.at[k], sem_in[k])
        pltpu.async_copy(
            table_hbm.at[idx_v.at[ci]],
            rows_v.at[k],
            sem_in[k],
        )

    def wait_in(k):
        pltpu.make_async_copy(pos_hbm.at[pl.ds(0, _C)], pos_v.at[k],
                              sem_in[k]).wait()
        pltpu.make_async_copy(table_hbm.at[pl.ds(0, _B * _C)], rows_v.at[k],
                              sem_in[k]).wait()

    def issue_out(ci, k):
        s0 = p0 + ci * _C
        for b in range(_B):
            pltpu.async_copy(
                rows_v.at[k, pl.ds(b * _C, _C)],
                out_hbm.at[b, pl.ds(s0, _C)],
                sem_out[k],
            )

    def wait_out(k):
        for _ in range(_B):
            pltpu.make_async_copy(pos_hbm.at[pl.ds(0, _C)], pos_v.at[k],
                                  sem_out[k]).wait()

    def add_chunk(k):
        @plsc.parallel_loop(0, _C * (_D // _LANES), unroll=4)
        def _(t):
            c = t >> 6
            off = (t & (_D // _LANES - 1)) * _LANES
            pv = pos_v[k, c, pl.ds(off, _LANES)]
            for b in range(_B):
                r = b * _C + c
                rows_v[k, r, pl.ds(off, _LANES)] = (
                    rows_v[k, r, pl.ds(off, _LANES)] + pv
                )

    for k in range(_NBUF - 1):
        issue_in(k, k)

    def ring(i, carry):
        for k in range(_NBUF):
            ci = i * _NBUF + k
            kprev = (k + _NBUF - 1) % _NBUF
            if k == 0:
                @pl.when(i < _NOUT - 1)
                def _():
                    @pl.when(i > 0)
                    def _():
                        wait_out(kprev)
                    issue_in(ci + _NBUF - 1, kprev)
            else:
                @pl.when(ci + _NBUF - 1 < _NCHUNK)
                def _():
                    wait_out(kprev)
                    issue_in(ci + _NBUF - 1, kprev)

            @pl.when(ci < _NCHUNK)
            def _():
                wait_in(k)
                add_chunk(k)
                issue_out(ci, k)
        return carry

    lax.fori_loop(0, _NOUT, ring, 0)
    for k in range(_NBUF):
        wait_out(k)


@jax.jit
def _emb(tokens, table, pos2d):
    mesh = plsc.VectorSubcoreMesh(core_axis_name="c", subcore_axis_name="s")
    return pl.kernel(
        _body,
        out_type=jax.ShapeDtypeStruct((_B, _S, _D), jnp.float32),
        mesh=mesh,
        compiler_params=pltpu.CompilerParams(skip_device_barrier=True),
        scratch_types=[
            pltpu.VMEM((_NCHUNK, _B * _C), jnp.int32),
            pltpu.VMEM((_NBUF, _C, _D), jnp.float32),
            pltpu.VMEM((_NBUF, _B * _C, _D), jnp.float32),
        ] + [pltpu.SemaphoreType.DMA] * (2 * _NBUF),
    )(tokens, table, pos2d)


def kernel(tokens, table, pos):
    tk = tokens.astype(jnp.int32).reshape(_B, _S // _C, _C)
    tk = tk.transpose(1, 0, 2).reshape(_S // _C, _B * _C)
    pos2d = pos.reshape(pos.shape[1], pos.shape[2])[: tokens.shape[1]]
    return _emb(tk, table, pos2d)


# speedup vs baseline: 6.0365x; 1.0014x over previous
import jax
import jax.numpy as jnp
from jax import lax
from jax.experimental import pallas as pl
from jax.experimental.pallas import tpu as pltpu
from jax.experimental.pallas import tpu_sc as plsc

_B = 4
_S = 4096
_D = 1024
_NC = 2
_NS = 16
_NW = _NC * _NS
_PPW = _S // _NW
_C = 8
_NCHUNK = _PPW // _C
_NBUF = 3
_NOUT = -(-_NCHUNK // _NBUF)
_LANES = 16


def _body(tokens_hbm, table_hbm, pos_hbm, out_hbm, idx_v, pos_v, rows_v,
          si0, si1, si2, so0, so1, so2):
    wid = lax.axis_index("s") * _NC + lax.axis_index("c")
    p0 = wid * _PPW
    sem_in = (si0, si1, si2)
    sem_out = (so0, so1, so2)

    pltpu.sync_copy(tokens_hbm.at[pl.ds(wid * _NCHUNK, _NCHUNK)], idx_v)

    def issue_in(ci, k):
        s0 = p0 + ci * _C
        pltpu.async_copy(pos_hbm.at[pl.ds(s0, _C)], pos_v.at[k], sem_in[k])
        pltpu.async_copy(
            table_hbm.at[idx_v.at[ci]],
            rows_v.at[k],
            sem_in[k],
        )

    def wait_in(k):
        pltpu.make_async_copy(pos_hbm.at[pl.ds(0, _C)], pos_v.at[k],
                              sem_in[k]).wait()
        pltpu.make_async_copy(table_hbm.at[pl.ds(0, _B * _C)], rows_v.at[k],
                              sem_in[k]).wait()

    def issue_out(ci, k):
        s0 = p0 + ci * _C
        for b in range(_B):
            pltpu.async_copy(
                rows_v.at[k, pl.ds(b * _C, _C)],
                out_hbm.at[b, pl.ds(s0, _C)],
                sem_out[k],
            )

    def wait_out(k):
        for _ in range(_B):
            pltpu.make_async_copy(pos_hbm.at[pl.ds(0, _C)], pos_v.at[k],
                                  sem_out[k]).wait()

    def add_chunk(k):
        @plsc.parallel_loop(0, _C * (_D // _LANES), unroll=4)
        def _(t):
            c = t >> 6
            off = (t & (_D // _LANES - 1)) * _LANES
            pv = pos_v[k, c, pl.ds(off, _LANES)]
            for b in range(_B):
                r = b * _C + c
                rows_v[k, r, pl.ds(off, _LANES)] = (
                    rows_v[k, r, pl.ds(off, _LANES)] + pv
                )

    for k in range(_NBUF - 1):
        issue_in(k, k)

    def ring(i, carry):
        for k in range(_NBUF):
            ci = i * _NBUF + k
            kprev = (k + _NBUF - 1) % _NBUF
            if k == 0:
                @pl.when(i < _NOUT - 1)
                def _():
                    @pl.when(i > 0)
                    def _():
                        wait_out(kprev)
                    issue_in(ci + _NBUF - 1, kprev)
            else:
                @pl.when(ci + _NBUF - 1 < _NCHUNK)
                def _():
                    wait_out(kprev)
                    issue_in(ci + _NBUF - 1, kprev)

            @pl.when(ci < _NCHUNK)
            def _():
                wait_in(k)
                add_chunk(k)
                issue_out(ci, k)
        return carry

    lax.fori_loop(0, _NOUT, ring, 0)
    for k in range(_NBUF):
        wait_out(k)


@jax.jit
def _emb(tokens, table, pos2d):
    mesh = plsc.VectorSubcoreMesh(core_axis_name="c", subcore_axis_name="s")
    return pl.kernel(
        _body,
        out_type=jax.ShapeDtypeStruct((_B, _S, _D), jnp.float32),
        mesh=mesh,
        scratch_types=[
            pltpu.VMEM((_NCHUNK, _B * _C), jnp.int32),
            pltpu.VMEM((_NBUF, _C, _D), jnp.float32),
            pltpu.VMEM((_NBUF, _B * _C, _D), jnp.float32),
        ] + [pltpu.SemaphoreType.DMA] * (2 * _NBUF),
    )(tokens, table, pos2d)


def kernel(tokens, table, pos):
    tk = tokens.astype(jnp.int32).reshape(_B, _S // _C, _C)
    tk = tk.transpose(1, 0, 2).reshape(_S // _C, _B * _C)
    pos2d = pos.reshape(pos.shape[1], pos.shape[2])[: tokens.shape[1]]
    return _emb(tk, table, pos2d)
